# baseline (speedup 1.0000x reference)
import functools

import jax
import jax.numpy as jnp
from jax import lax
from jax.experimental import pallas as pl
from jax.experimental.pallas import tpu as pltpu
from jax.experimental.pallas import tpu_sc as plsc


@functools.lru_cache(maxsize=None)
def _make_pe_gather(B, V, D):
    info = plsc.get_sparse_core_info()
    NC, NS, L = info.num_cores, info.num_subcores, info.num_lanes
    NW = NC * NS
    assert B % NW == 0 and D % L == 0
    b_per_w = B // NW
    CH = 32
    assert b_per_w % CH == 0
    NCH = b_per_w // CH
    NBUF = 5
    LEAD = NBUF - 1
    mesh = plsc.VectorSubcoreMesh(core_axis_name="c", subcore_axis_name="s")

    @functools.partial(
        pl.kernel,
        mesh=mesh,
        out_type=jax.ShapeDtypeStruct((B, D), jnp.float32),
        scratch_types=[
            pltpu.VMEM((b_per_w,), jnp.float32),
            pltpu.VMEM((NCH, CH), jnp.int32),
            pltpu.VMEM((NBUF, CH, D), jnp.float32),
        ]
        + [pltpu.SemaphoreType.DMA] * (2 * NBUF),
    )
    def k(t_hbm, pe_hbm, out_hbm, t_v, idx_v, rows_v, *sems):
        gsem = sems[:NBUF]
        osem = sems[NBUF:]
        wid = lax.axis_index("s") * NC + lax.axis_index("c")
        base = wid * b_per_w
        scale = jnp.float32(V - 1)

        def compute_idx(c):
            for j in range(CH // L):
                tv = t_v[pl.ds(c * CH + j * L, L)]
                iv = jnp.clip((tv * scale).astype(jnp.int32), 0, V - 1)
                idx_v[c, pl.ds(j * L, L)] = iv

        def gather(c):
            b = c % NBUF
            return pltpu.async_copy(pe_hbm.at[idx_v.at[c]], rows_v.at[b], gsem[b])

        def put(c):
            b = c % NBUF
            return pltpu.async_copy(
                rows_v.at[b], out_hbm.at[pl.ds(base + c * CH, CH)], osem[b])

        gpend = [None] * NBUF
        opend = [None] * NBUF
        pltpu.sync_copy(t_hbm.at[pl.ds(base, CH)], t_v.at[pl.ds(0, CH)])
        compute_idx(0)
        gpend[0] = gather(0)
        pltpu.sync_copy(t_hbm.at[pl.ds(base + CH, b_per_w - CH)],
                        t_v.at[pl.ds(CH, b_per_w - CH)])
        for i in range(1, NCH + LEAD):
            cg = i
            if cg < NCH:
                b = cg % NBUF
                if opend[b] is not None:
                    opend[b].wait()
                    opend[b] = None
                compute_idx(cg)
                gpend[b] = gather(cg)
            cp = i - LEAD
            if 0 <= cp < NCH:
                b = cp % NBUF
                gpend[b].wait()
                opend[b] = put(cp)
        for p in opend:
            if p is not None:
                p.wait()

    return k


def kernel(t, pe):
    B, = t.shape
    V, D = pe.shape
    return _make_pe_gather(B, V, D)(t, pe)


# speedup vs baseline: 1.0132x; 1.0132x over previous
import functools

import jax
import jax.numpy as jnp
from jax import lax
from jax.experimental import pallas as pl
from jax.experimental.pallas import tpu as pltpu
from jax.experimental.pallas import tpu_sc as plsc


@functools.lru_cache(maxsize=None)
def _make_pe_gather(B, V, D):
    info = plsc.get_sparse_core_info()
    NC, NS, L = info.num_cores, info.num_subcores, info.num_lanes
    NW = NC * NS
    assert B % NW == 0 and D % L == 0
    b_per_w = B // NW
    CH = 64
    assert b_per_w % CH == 0
    NCH = b_per_w // CH
    NBUF = 3
    LEAD = NBUF - 1
    mesh = plsc.VectorSubcoreMesh(core_axis_name="c", subcore_axis_name="s")

    @functools.partial(
        pl.kernel,
        mesh=mesh,
        out_type=jax.ShapeDtypeStruct((B, D), jnp.float32),
        scratch_types=[
            pltpu.VMEM((b_per_w,), jnp.float32),
            pltpu.VMEM((NCH, CH), jnp.int32),
            pltpu.VMEM((NBUF, CH, D), jnp.float32),
        ]
        + [pltpu.SemaphoreType.DMA] * (2 * NBUF),
    )
    def k(t_hbm, pe_hbm, out_hbm, t_v, idx_v, rows_v, *sems):
        gsem = sems[:NBUF]
        osem = sems[NBUF:]
        wid = lax.axis_index("s") * NC + lax.axis_index("c")
        base = wid * b_per_w
        scale = jnp.float32(V - 1)

        def compute_idx(c):
            for j in range(CH // L):
                tv = t_v[pl.ds(c * CH + j * L, L)]
                iv = jnp.clip((tv * scale).astype(jnp.int32), 0, V - 1)
                idx_v[c, pl.ds(j * L, L)] = iv

        def gather(c):
            b = c % NBUF
            return pltpu.async_copy(pe_hbm.at[idx_v.at[c]], rows_v.at[b], gsem[b])

        def put(c):
            b = c % NBUF
            return pltpu.async_copy(
                rows_v.at[b], out_hbm.at[pl.ds(base + c * CH, CH)], osem[b])

        gpend = [None] * NBUF
        opend = [None] * NBUF
        pltpu.sync_copy(t_hbm.at[pl.ds(base, CH)], t_v.at[pl.ds(0, CH)])
        compute_idx(0)
        gpend[0] = gather(0)
        pltpu.sync_copy(t_hbm.at[pl.ds(base + CH, b_per_w - CH)],
                        t_v.at[pl.ds(CH, b_per_w - CH)])
        for i in range(1, NCH + LEAD):
            cg = i
            if cg < NCH:
                b = cg % NBUF
                if opend[b] is not None:
                    opend[b].wait()
                    opend[b] = None
                compute_idx(cg)
                gpend[b] = gather(cg)
            cp = i - LEAD
            if 0 <= cp < NCH:
                b = cp % NBUF
                gpend[b].wait()
                opend[b] = put(cp)
        for p in opend:
            if p is not None:
                p.wait()

    return k


def kernel(t, pe):
    B, = t.shape
    V, D = pe.shape
    return _make_pe_gather(B, V, D)(t, pe)
